# baseline (speedup 1.0000x reference)
import functools

import jax
import jax.numpy as jnp
from jax import lax
from jax.experimental import pallas as pl
from jax.experimental.pallas import tpu as pltpu
from jax.experimental.pallas import tpu_sc as plsc

_N = 10000
_E = 160000
_HID = 128
_LAT = 64
_NC = 2
_NS = 16
_K = 125
_NCH = _E // (_NC * _NS * _K)
_NP = 10240
_RPT = _NP // _NS
_BM = 1000

def _sc_mesh():
    return plsc.VectorSubcoreMesh(core_axis_name="c", subcore_axis_name="s")


def _sc_deg(dst_r, ones128, zeros128):
    @functools.partial(
        pl.kernel,
        out_type=jax.ShapeDtypeStruct((_NC, _NP, _HID), jnp.float32),
        mesh=_sc_mesh(),
        scratch_types=[
            pltpu.VMEM((_NCH, _K), jnp.int32),
            pltpu.VMEM((_K, _HID), jnp.float32),
            pltpu.VMEM_SHARED((_NP, _HID), jnp.float32),
            pltpu.SemaphoreType.DMA,
        ],
    )
    def body(dst_hbm, ones_hbm, zeros_hbm, out_hbm, dst_v, ones_v, acc_sh, dsem):
        c = lax.axis_index("c")
        s = lax.axis_index("s")
        rows = pl.ds(s * _RPT, _RPT)
        pltpu.sync_copy(zeros_hbm, acc_sh.at[rows])
        pltpu.sync_copy(dst_hbm.at[c, s], dst_v)
        pltpu.sync_copy(ones_hbm, ones_v)
        plsc.subcore_barrier()
        hs = [pltpu.async_copy(ones_v, acc_sh.at[dst_v.at[j]], dsem, add=True)
              for j in range(_NCH)]
        for h in hs:
            h.wait()
        plsc.subcore_barrier()
        pltpu.sync_copy(acc_sh.at[rows], out_hbm.at[c, rows])

    return body(dst_r, ones128, zeros128)


def _sc_prop(u, src_r, dst_r, zeros128):
    @functools.partial(
        pl.kernel,
        out_type=jax.ShapeDtypeStruct((_NC, _NP, _HID), jnp.float32),
        mesh=_sc_mesh(),
        scratch_types=[
            pltpu.VMEM((_NCH, _K), jnp.int32),
            pltpu.VMEM((_NCH, _K), jnp.int32),
            pltpu.VMEM((2, _K, _HID), jnp.float32),
            pltpu.VMEM_SHARED((_NP, _HID), jnp.float32),
            pltpu.SemaphoreType.DMA,
            pltpu.SemaphoreType.DMA,
            pltpu.SemaphoreType.DMA,
            pltpu.SemaphoreType.DMA,
        ],
    )
    def body(u_hbm, src_hbm, dst_hbm, zeros_hbm, out_hbm,
             src_v, dst_v, rows_v, acc_sh, g0, g1, s0, s1):
        c = lax.axis_index("c")
        s = lax.axis_index("s")
        rows = pl.ds(s * _RPT, _RPT)
        pltpu.sync_copy(zeros_hbm, acc_sh.at[rows])
        pltpu.sync_copy(src_hbm.at[c, s], src_v)
        pltpu.sync_copy(dst_hbm.at[c, s], dst_v)
        plsc.subcore_barrier()
        gs = (g0, g1)
        ss = (s0, s1)
        gh, sh = {}, {}

        def fire_gather(idx):
            b = idx % 2
            gh[idx] = pltpu.async_copy(u_hbm.at[src_v.at[idx]], rows_v.at[b], gs[b])

        fire_gather(0)
        for j in range(_NCH):
            b = j % 2
            gh[j].wait()
            sh[j] = pltpu.async_copy(rows_v.at[b], acc_sh.at[dst_v.at[j]],
                                     ss[b], add=True)
            if j + 1 < _NCH:
                if j >= 1:
                    sh[j - 1].wait()
                fire_gather(j + 1)
        sh[_NCH - 1].wait()
        sh[_NCH - 2].wait()
        plsc.subcore_barrier()
        pltpu.sync_copy(acc_sh.at[rows], out_hbm.at[c, rows])

    return body(u, src_r, dst_r, zeros128)


def _tc_embed(x, pos, emb, w1a, w1b):
    def body(x_ref, pos_ref, emb_ref, w1a_ref, w1b_ref, t_ref):
        xi = x_ref[...]
        oh = (xi == lax.broadcasted_iota(jnp.int32, (_BM, 128), 1)).astype(jnp.float32)
        hemb = jnp.dot(oh, emb_ref[...], preferred_element_type=jnp.float32)
        t = jnp.dot(hemb, w1a_ref[...], preferred_element_type=jnp.float32)
        p = pos_ref[...]
        w1b = w1b_ref[...]
        t = t + p[:, 0:1] * w1b[0:1, :] + p[:, 1:2] * w1b[1:2, :] + p[:, 2:3] * w1b[2:3, :]
        t_ref[...] = t

    return pl.pallas_call(
        body,
        grid=(_N // _BM,),
        in_specs=[
            pl.BlockSpec((_BM, 1), lambda i: (i, 0)),
            pl.BlockSpec((_BM, 3), lambda i: (i, 0)),
            pl.BlockSpec((128, _HID), lambda i: (0, 0)),
            pl.BlockSpec((_HID, _HID), lambda i: (0, 0)),
            pl.BlockSpec((3, _HID), lambda i: (0, 0)),
        ],
        out_specs=pl.BlockSpec((_BM, _HID), lambda i: (i, 0)),
        out_shape=jax.ShapeDtypeStruct((_N, _HID), jnp.float32),
    )(x, pos, emb, w1a, w1b)


def _tc_scale(t, degp):
    def body(t_ref, degp_ref, u_ref, dinv_ref):
        deg = 1.0 + degp_ref[0, :, 0:1] + degp_ref[1, :, 0:1]
        dinv = lax.rsqrt(deg)
        u_ref[...] = dinv * t_ref[...]
        dinv_ref[...] = jnp.broadcast_to(dinv, (_BM, 16))

    return pl.pallas_call(
        body,
        grid=(_N // _BM,),
        in_specs=[
            pl.BlockSpec((_BM, _HID), lambda i: (i, 0)),
            pl.BlockSpec((_NC, _BM, _HID), lambda i: (0, i, 0)),
        ],
        out_specs=[
            pl.BlockSpec((_BM, _HID), lambda i: (i, 0)),
            pl.BlockSpec((_BM, 16), lambda i: (i, 0)),
        ],
        out_shape=[
            jax.ShapeDtypeStruct((_N, _HID), jnp.float32),
            jax.ShapeDtypeStruct((_N, 16), jnp.float32),
        ],
    )(t, degp)


def _tc_combine(a, u, dinv16, w, b):
    def body(a_ref, u_ref, dinv_ref, w_ref, b_ref, out_ref):
        dinv = dinv_ref[:, 0:1]
        h = jnp.maximum(dinv * (a_ref[0] + a_ref[1] + u_ref[...]) + b_ref[...], 0.0)
        t = jnp.dot(h, w_ref[...], preferred_element_type=jnp.float32)
        out_ref[...] = dinv * t

    return pl.pallas_call(
        body,
        grid=(_N // _BM,),
        in_specs=[
            pl.BlockSpec((_NC, _BM, _HID), lambda i: (0, i, 0)),
            pl.BlockSpec((_BM, _HID), lambda i: (i, 0)),
            pl.BlockSpec((_BM, 16), lambda i: (i, 0)),
            pl.BlockSpec((_HID, _HID), lambda i: (0, 0)),
            pl.BlockSpec((1, _HID), lambda i: (0, 0)),
        ],
        out_specs=pl.BlockSpec((_BM, _HID), lambda i: (i, 0)),
        out_shape=jax.ShapeDtypeStruct((_N, _HID), jnp.float32),
    )(a, u, dinv16, w, b)


def _tc_combine_nw(a, u, dinv16, b):
    def body(a_ref, u_ref, dinv_ref, b_ref, out_ref):
        dinv = dinv_ref[:, 0:1]
        h = jnp.maximum(dinv * (a_ref[0] + a_ref[1] + u_ref[...]) + b_ref[...], 0.0)
        out_ref[...] = dinv * h

    return pl.pallas_call(
        body,
        grid=(_N // _BM,),
        in_specs=[
            pl.BlockSpec((_NC, _BM, _HID), lambda i: (0, i, 0)),
            pl.BlockSpec((_BM, _HID), lambda i: (i, 0)),
            pl.BlockSpec((_BM, 16), lambda i: (i, 0)),
            pl.BlockSpec((1, _HID), lambda i: (0, 0)),
        ],
        out_specs=pl.BlockSpec((_BM, _HID), lambda i: (i, 0)),
        out_shape=jax.ShapeDtypeStruct((_N, _HID), jnp.float32),
    )(a, u, dinv16, b)


def _tc_heads(a, u, dinv16, wmu, bmu, wls, bls, eps):
    def body(a_ref, u_ref, dinv_ref, wmu_ref, bmu_ref, wls_ref, bls_ref, eps_ref,
             mu_ref, ls_ref, z_ref):
        dinv = dinv_ref[:, 0:1]
        sfeat = dinv * (a_ref[0] + a_ref[1] + u_ref[...])
        mu = jnp.dot(sfeat, wmu_ref[...], preferred_element_type=jnp.float32) + bmu_ref[...]
        ls = jnp.dot(sfeat, wls_ref[...], preferred_element_type=jnp.float32) + bls_ref[...]
        mu_ref[...] = mu
        ls_ref[...] = ls
        z_ref[...] = (mu + eps_ref[...] * jnp.exp(0.5 * ls)).astype(jnp.bfloat16)

    return pl.pallas_call(
        body,
        grid=(_N // _BM,),
        in_specs=[
            pl.BlockSpec((_NC, _BM, _HID), lambda i: (0, i, 0)),
            pl.BlockSpec((_BM, _HID), lambda i: (i, 0)),
            pl.BlockSpec((_BM, 16), lambda i: (i, 0)),
            pl.BlockSpec((_HID, _LAT), lambda i: (0, 0)),
            pl.BlockSpec((1, _LAT), lambda i: (0, 0)),
            pl.BlockSpec((_HID, _LAT), lambda i: (0, 0)),
            pl.BlockSpec((1, _LAT), lambda i: (0, 0)),
            pl.BlockSpec((_BM, _LAT), lambda i: (i, 0)),
        ],
        out_specs=[
            pl.BlockSpec((_BM, _LAT), lambda i: (i, 0)),
            pl.BlockSpec((_BM, _LAT), lambda i: (i, 0)),
            pl.BlockSpec((_BM, _LAT), lambda i: (i, 0)),
        ],
        out_shape=[
            jax.ShapeDtypeStruct((_N, _LAT), jnp.float32),
            jax.ShapeDtypeStruct((_N, _LAT), jnp.float32),
            jax.ShapeDtypeStruct((_N, _LAT), jnp.bfloat16),
        ],
    )(a, u, dinv16, wmu, bmu, wls, bls, eps)


def _tc_zz(z):
    def body(zi_ref, zj_ref, out_ref):
        out_ref[...] = lax.dot_general(
            zi_ref[...], zj_ref[...], (((1,), (1,)), ((), ())),
            preferred_element_type=jnp.float32)

    bz = 1024
    ng = pl.cdiv(_N, bz)
    return pl.pallas_call(
        body,
        grid=(ng, ng),
        in_specs=[
            pl.BlockSpec((bz, _LAT), lambda i, j: (i, 0)),
            pl.BlockSpec((bz, _LAT), lambda i, j: (j, 0)),
        ],
        out_specs=pl.BlockSpec((bz, bz), lambda i, j: (i, j)),
        out_shape=jax.ShapeDtypeStruct((_N, _N), jnp.float32),
    )(z, z)


def kernel(x, pos, edge_index, emb, W1, b1, W2, b2, Wmu, bmu, Wls, bls):
    src_r = edge_index[0].reshape(_NC, _NS, _NCH, _K)
    dst_r = edge_index[1].reshape(_NC, _NS, _NCH, _K)
    ones128 = jnp.ones((_K, _HID), jnp.float32)
    zeros128 = jnp.zeros((_RPT, _HID), jnp.float32)
    w1a = W1[:_HID]
    w1b = W1[_HID:]
    emb_p = jnp.pad(emb, ((0, 128 - emb.shape[0]), (0, 0)))
    eps = jax.random.normal(jax.random.key(42), (_N, _LAT), jnp.float32)

    degp = _sc_deg(dst_r, ones128, zeros128)
    t1 = _tc_embed(x, pos, emb_p, w1a, w1b)
    u1, dinv16 = _tc_scale(t1, degp)
    a1 = _sc_prop(u1, src_r, dst_r, zeros128)
    u2 = _tc_combine(a1, u1, dinv16, W2, b1.reshape(1, _HID))
    a2 = _sc_prop(u2, src_r, dst_r, zeros128)
    u3 = _tc_combine_nw(a2, u2, dinv16, b2.reshape(1, _HID))
    a3 = _sc_prop(u3, src_r, dst_r, zeros128)
    mu, ls, z = _tc_heads(a3, u3, dinv16, Wmu, bmu.reshape(1, _LAT),
                          Wls, bls.reshape(1, _LAT), eps)
    adj = _tc_zz(z)
    return adj, mu, ls


# speedup vs baseline: 1.0009x; 1.0009x over previous
import functools

import jax
import jax.numpy as jnp
from jax import lax
from jax.experimental import pallas as pl
from jax.experimental.pallas import tpu as pltpu
from jax.experimental.pallas import tpu_sc as plsc

_N = 10000
_E = 160000
_HID = 128
_LAT = 64
_NC = 2
_NS = 16
_K = 125
_NCH = _E // (_NC * _NS * _K)
_NP = 10240
_RPT = _NP // _NS
_BM = 1000

def _sc_mesh():
    return plsc.VectorSubcoreMesh(core_axis_name="c", subcore_axis_name="s")


def _sc_deg(dst_r, ones128, zeros128):
    @functools.partial(
        pl.kernel,
        out_type=jax.ShapeDtypeStruct((_NC, _NP, _HID), jnp.float32),
        mesh=_sc_mesh(),
        scratch_types=[
            pltpu.VMEM((_NCH, _K), jnp.int32),
            pltpu.VMEM((_K, _HID), jnp.float32),
            pltpu.VMEM_SHARED((_NP, _HID), jnp.float32),
            pltpu.SemaphoreType.DMA,
        ],
    )
    def body(dst_hbm, ones_hbm, zeros_hbm, out_hbm, dst_v, ones_v, acc_sh, dsem):
        c = lax.axis_index("c")
        s = lax.axis_index("s")
        rows = pl.ds(s * _RPT, _RPT)
        pltpu.sync_copy(zeros_hbm, acc_sh.at[rows])
        pltpu.sync_copy(dst_hbm.at[c, s], dst_v)
        pltpu.sync_copy(ones_hbm, ones_v)
        plsc.subcore_barrier()
        hs = []
        for j in range(_NCH):
            hs.append(pltpu.async_copy(ones_v, acc_sh.at[dst_v.at[j]], dsem, add=True))
            if j >= 2:
                hs[j - 2].wait()
        hs[_NCH - 1].wait()
        hs[_NCH - 2].wait()
        plsc.subcore_barrier()
        pltpu.sync_copy(acc_sh.at[rows], out_hbm.at[c, rows])

    return body(dst_r, ones128, zeros128)


def _sc_prop(u, src_r, dst_r, zeros128):
    @functools.partial(
        pl.kernel,
        out_type=jax.ShapeDtypeStruct((_NC, _NP, _HID), jnp.float32),
        mesh=_sc_mesh(),
        scratch_types=[
            pltpu.VMEM((_NCH, _K), jnp.int32),
            pltpu.VMEM((_NCH, _K), jnp.int32),
            pltpu.VMEM((2, _K, _HID), jnp.float32),
            pltpu.VMEM_SHARED((_NP, _HID), jnp.float32),
            pltpu.SemaphoreType.DMA,
            pltpu.SemaphoreType.DMA,
            pltpu.SemaphoreType.DMA,
            pltpu.SemaphoreType.DMA,
        ],
    )
    def body(u_hbm, src_hbm, dst_hbm, zeros_hbm, out_hbm,
             src_v, dst_v, rows_v, acc_sh, g0, g1, s0, s1):
        c = lax.axis_index("c")
        s = lax.axis_index("s")
        rows = pl.ds(s * _RPT, _RPT)
        pltpu.sync_copy(zeros_hbm, acc_sh.at[rows])
        pltpu.sync_copy(src_hbm.at[c, s], src_v)
        pltpu.sync_copy(dst_hbm.at[c, s], dst_v)
        plsc.subcore_barrier()
        gs = (g0, g1)
        ss = (s0, s1)
        gh, sh = {}, {}

        def fire_gather(idx):
            b = idx % 2
            gh[idx] = pltpu.async_copy(u_hbm.at[src_v.at[idx]], rows_v.at[b], gs[b])

        fire_gather(0)
        for j in range(_NCH):
            b = j % 2
            gh[j].wait()
            sh[j] = pltpu.async_copy(rows_v.at[b], acc_sh.at[dst_v.at[j]],
                                     ss[b], add=True)
            if j + 1 < _NCH:
                if j >= 1:
                    sh[j - 1].wait()
                fire_gather(j + 1)
        sh[_NCH - 1].wait()
        sh[_NCH - 2].wait()
        plsc.subcore_barrier()
        pltpu.sync_copy(acc_sh.at[rows], out_hbm.at[c, rows])

    return body(u, src_r, dst_r, zeros128)


def _tc_embed(x, pos, emb, w1a, w1b):
    def body(x_ref, pos_ref, emb_ref, w1a_ref, w1b_ref, t_ref):
        xi = x_ref[...]
        oh = (xi == lax.broadcasted_iota(jnp.int32, (_BM, 128), 1)).astype(jnp.float32)
        hemb = jnp.dot(oh, emb_ref[...], preferred_element_type=jnp.float32)
        t = jnp.dot(hemb, w1a_ref[...], preferred_element_type=jnp.float32)
        p = pos_ref[...]
        w1b = w1b_ref[...]
        t = t + p[:, 0:1] * w1b[0:1, :] + p[:, 1:2] * w1b[1:2, :] + p[:, 2:3] * w1b[2:3, :]
        t_ref[...] = t

    return pl.pallas_call(
        body,
        grid=(_N // _BM,),
        in_specs=[
            pl.BlockSpec((_BM, 1), lambda i: (i, 0)),
            pl.BlockSpec((_BM, 3), lambda i: (i, 0)),
            pl.BlockSpec((128, _HID), lambda i: (0, 0)),
            pl.BlockSpec((_HID, _HID), lambda i: (0, 0)),
            pl.BlockSpec((3, _HID), lambda i: (0, 0)),
        ],
        out_specs=pl.BlockSpec((_BM, _HID), lambda i: (i, 0)),
        out_shape=jax.ShapeDtypeStruct((_N, _HID), jnp.float32),
    )(x, pos, emb, w1a, w1b)


def _tc_scale(t, degp):
    def body(t_ref, degp_ref, u_ref, dinv_ref):
        deg = 1.0 + degp_ref[0, :, 0:1] + degp_ref[1, :, 0:1]
        dinv = lax.rsqrt(deg)
        u_ref[...] = dinv * t_ref[...]
        dinv_ref[...] = jnp.broadcast_to(dinv, (_BM, 16))

    return pl.pallas_call(
        body,
        grid=(_N // _BM,),
        in_specs=[
            pl.BlockSpec((_BM, _HID), lambda i: (i, 0)),
            pl.BlockSpec((_NC, _BM, _HID), lambda i: (0, i, 0)),
        ],
        out_specs=[
            pl.BlockSpec((_BM, _HID), lambda i: (i, 0)),
            pl.BlockSpec((_BM, 16), lambda i: (i, 0)),
        ],
        out_shape=[
            jax.ShapeDtypeStruct((_N, _HID), jnp.float32),
            jax.ShapeDtypeStruct((_N, 16), jnp.float32),
        ],
    )(t, degp)


def _tc_combine(a, u, dinv16, w, b):
    def body(a_ref, u_ref, dinv_ref, w_ref, b_ref, out_ref):
        dinv = dinv_ref[:, 0:1]
        h = jnp.maximum(dinv * (a_ref[0] + a_ref[1] + u_ref[...]) + b_ref[...], 0.0)
        t = jnp.dot(h, w_ref[...], preferred_element_type=jnp.float32)
        out_ref[...] = dinv * t

    return pl.pallas_call(
        body,
        grid=(_N // _BM,),
        in_specs=[
            pl.BlockSpec((_NC, _BM, _HID), lambda i: (0, i, 0)),
            pl.BlockSpec((_BM, _HID), lambda i: (i, 0)),
            pl.BlockSpec((_BM, 16), lambda i: (i, 0)),
            pl.BlockSpec((_HID, _HID), lambda i: (0, 0)),
            pl.BlockSpec((1, _HID), lambda i: (0, 0)),
        ],
        out_specs=pl.BlockSpec((_BM, _HID), lambda i: (i, 0)),
        out_shape=jax.ShapeDtypeStruct((_N, _HID), jnp.float32),
    )(a, u, dinv16, w, b)


def _tc_combine_nw(a, u, dinv16, b):
    def body(a_ref, u_ref, dinv_ref, b_ref, out_ref):
        dinv = dinv_ref[:, 0:1]
        h = jnp.maximum(dinv * (a_ref[0] + a_ref[1] + u_ref[...]) + b_ref[...], 0.0)
        out_ref[...] = dinv * h

    return pl.pallas_call(
        body,
        grid=(_N // _BM,),
        in_specs=[
            pl.BlockSpec((_NC, _BM, _HID), lambda i: (0, i, 0)),
            pl.BlockSpec((_BM, _HID), lambda i: (i, 0)),
            pl.BlockSpec((_BM, 16), lambda i: (i, 0)),
            pl.BlockSpec((1, _HID), lambda i: (0, 0)),
        ],
        out_specs=pl.BlockSpec((_BM, _HID), lambda i: (i, 0)),
        out_shape=jax.ShapeDtypeStruct((_N, _HID), jnp.float32),
    )(a, u, dinv16, b)


def _tc_heads(a, u, dinv16, wmu, bmu, wls, bls, eps):
    def body(a_ref, u_ref, dinv_ref, wmu_ref, bmu_ref, wls_ref, bls_ref, eps_ref,
             mu_ref, ls_ref, z_ref):
        dinv = dinv_ref[:, 0:1]
        sfeat = dinv * (a_ref[0] + a_ref[1] + u_ref[...])
        mu = jnp.dot(sfeat, wmu_ref[...], preferred_element_type=jnp.float32) + bmu_ref[...]
        ls = jnp.dot(sfeat, wls_ref[...], preferred_element_type=jnp.float32) + bls_ref[...]
        mu_ref[...] = mu
        ls_ref[...] = ls
        z_ref[...] = (mu + eps_ref[...] * jnp.exp(0.5 * ls)).astype(jnp.bfloat16)

    return pl.pallas_call(
        body,
        grid=(_N // _BM,),
        in_specs=[
            pl.BlockSpec((_NC, _BM, _HID), lambda i: (0, i, 0)),
            pl.BlockSpec((_BM, _HID), lambda i: (i, 0)),
            pl.BlockSpec((_BM, 16), lambda i: (i, 0)),
            pl.BlockSpec((_HID, _LAT), lambda i: (0, 0)),
            pl.BlockSpec((1, _LAT), lambda i: (0, 0)),
            pl.BlockSpec((_HID, _LAT), lambda i: (0, 0)),
            pl.BlockSpec((1, _LAT), lambda i: (0, 0)),
            pl.BlockSpec((_BM, _LAT), lambda i: (i, 0)),
        ],
        out_specs=[
            pl.BlockSpec((_BM, _LAT), lambda i: (i, 0)),
            pl.BlockSpec((_BM, _LAT), lambda i: (i, 0)),
            pl.BlockSpec((_BM, _LAT), lambda i: (i, 0)),
        ],
        out_shape=[
            jax.ShapeDtypeStruct((_N, _LAT), jnp.float32),
            jax.ShapeDtypeStruct((_N, _LAT), jnp.float32),
            jax.ShapeDtypeStruct((_N, _LAT), jnp.bfloat16),
        ],
    )(a, u, dinv16, wmu, bmu, wls, bls, eps)


def _tc_zz(z):
    def body(zi_ref, zj_ref, out_ref):
        out_ref[...] = lax.dot_general(
            zi_ref[...], zj_ref[...], (((1,), (1,)), ((), ())),
            preferred_element_type=jnp.float32)

    bz = 1024
    ng = pl.cdiv(_N, bz)
    return pl.pallas_call(
        body,
        grid=(ng, ng),
        in_specs=[
            pl.BlockSpec((bz, _LAT), lambda i, j: (i, 0)),
            pl.BlockSpec((bz, _LAT), lambda i, j: (j, 0)),
        ],
        out_specs=pl.BlockSpec((bz, bz), lambda i, j: (i, j)),
        out_shape=jax.ShapeDtypeStruct((_N, _N), jnp.float32),
    )(z, z)


def kernel(x, pos, edge_index, emb, W1, b1, W2, b2, Wmu, bmu, Wls, bls):
    src_r = edge_index[0].reshape(_NC, _NS, _NCH, _K)
    dst_r = edge_index[1].reshape(_NC, _NS, _NCH, _K)
    ones128 = jnp.ones((_K, _HID), jnp.float32)
    zeros128 = jnp.zeros((_RPT, _HID), jnp.float32)
    w1a = W1[:_HID]
    w1b = W1[_HID:]
    emb_p = jnp.pad(emb, ((0, 128 - emb.shape[0]), (0, 0)))
    eps = jax.random.normal(jax.random.key(42), (_N, _LAT), jnp.float32)

    degp = _sc_deg(dst_r, ones128, zeros128)
    t1 = _tc_embed(x, pos, emb_p, w1a, w1b)
    u1, dinv16 = _tc_scale(t1, degp)
    a1 = _sc_prop(u1, src_r, dst_r, zeros128)
    u2 = _tc_combine(a1, u1, dinv16, W2, b1.reshape(1, _HID))
    a2 = _sc_prop(u2, src_r, dst_r, zeros128)
    u3 = _tc_combine_nw(a2, u2, dinv16, b2.reshape(1, _HID))
    a3 = _sc_prop(u3, src_r, dst_r, zeros128)
    mu, ls, z = _tc_heads(a3, u3, dinv16, Wmu, bmu.reshape(1, _LAT),
                          Wls, bls.reshape(1, _LAT), eps)
    adj = _tc_zz(z)
    return adj, mu, ls


# speedup vs baseline: 1.1468x; 1.1458x over previous
import functools

import jax
import jax.numpy as jnp
from jax import lax
from jax.experimental import pallas as pl
from jax.experimental.pallas import tpu as pltpu
from jax.experimental.pallas import tpu_sc as plsc

_N = 10000
_E = 160000
_HID = 128
_LAT = 64
_NC = 2
_NS = 16
_K = 125
_NCH = _E // (_NC * _NS * _K)
_NP = 10240
_RPT = _NP // _NS
_BM = 1000

def _sc_mesh():
    return plsc.VectorSubcoreMesh(core_axis_name="c", subcore_axis_name="s")


def _sc_deg(dst_r, ones128, zeros128):
    @functools.partial(
        pl.kernel,
        out_type=jax.ShapeDtypeStruct((_NC, _NP, _HID), jnp.float32),
        mesh=_sc_mesh(),
        scratch_types=[
            pltpu.VMEM((_NCH, _K), jnp.int32),
            pltpu.VMEM((_K, _HID), jnp.float32),
            pltpu.VMEM_SHARED((_NP, _HID), jnp.float32),
            pltpu.SemaphoreType.DMA,
        ],
    )
    def body(dst_hbm, ones_hbm, zeros_hbm, out_hbm, dst_v, ones_v, acc_sh, dsem):
        c = lax.axis_index("c")
        s = lax.axis_index("s")
        rows = pl.ds(s * _RPT, _RPT)
        pltpu.sync_copy(zeros_hbm, acc_sh.at[rows])
        pltpu.sync_copy(dst_hbm.at[c, s], dst_v)
        pltpu.sync_copy(ones_hbm, ones_v)
        plsc.subcore_barrier()
        for j in range(_NCH):
            pltpu.sync_copy(ones_v, acc_sh.at[dst_v.at[j]], add=True)
        plsc.subcore_barrier()
        pltpu.sync_copy(acc_sh.at[rows], out_hbm.at[c, rows])

    return body(dst_r, ones128, zeros128)


def _sc_prop(u, src_r, dst_r, zeros128):
    @functools.partial(
        pl.kernel,
        out_type=jax.ShapeDtypeStruct((_NC, _NP, _HID), jnp.float32),
        mesh=_sc_mesh(),
        scratch_types=[
            pltpu.VMEM((_NCH, _K), jnp.int32),
            pltpu.VMEM((_NCH, _K), jnp.int32),
            pltpu.VMEM((2, _K, _HID), jnp.float32),
            pltpu.VMEM_SHARED((_NP, _HID), jnp.float32),
            pltpu.SemaphoreType.DMA,
            pltpu.SemaphoreType.DMA,
            pltpu.SemaphoreType.DMA,
            pltpu.SemaphoreType.DMA,
        ],
    )
    def body(u_hbm, src_hbm, dst_hbm, zeros_hbm, out_hbm,
             src_v, dst_v, rows_v, acc_sh, g0, g1, s0, s1):
        c = lax.axis_index("c")
        s = lax.axis_index("s")
        rows = pl.ds(s * _RPT, _RPT)
        pltpu.sync_copy(zeros_hbm, acc_sh.at[rows])
        pltpu.sync_copy(src_hbm.at[c, s], src_v)
        pltpu.sync_copy(dst_hbm.at[c, s], dst_v)
        plsc.subcore_barrier()
        gs = (g0, g1)
        gh = {}

        def fire_gather(idx):
            b = idx % 2
            gh[idx] = pltpu.async_copy(u_hbm.at[src_v.at[idx]], rows_v.at[b], gs[b])

        fire_gather(0)
        for j in range(_NCH):
            b = j % 2
            if j + 1 < _NCH:
                fire_gather(j + 1)
            gh[j].wait()
            pltpu.sync_copy(rows_v.at[b], acc_sh.at[dst_v.at[j]], add=True)
        plsc.subcore_barrier()
        pltpu.sync_copy(acc_sh.at[rows], out_hbm.at[c, rows])

    return body(u, src_r, dst_r, zeros128)


def _tc_embed(x, pos, emb, w1a, w1b):
    def body(x_ref, pos_ref, emb_ref, w1a_ref, w1b_ref, t_ref):
        xi = x_ref[...]
        oh = (xi == lax.broadcasted_iota(jnp.int32, (_BM, 128), 1)).astype(jnp.float32)
        hemb = jnp.dot(oh, emb_ref[...], preferred_element_type=jnp.float32)
        t = jnp.dot(hemb, w1a_ref[...], preferred_element_type=jnp.float32)
        p = pos_ref[...]
        w1b = w1b_ref[...]
        t = t + p[:, 0:1] * w1b[0:1, :] + p[:, 1:2] * w1b[1:2, :] + p[:, 2:3] * w1b[2:3, :]
        t_ref[...] = t

    return pl.pallas_call(
        body,
        grid=(_N // _BM,),
        in_specs=[
            pl.BlockSpec((_BM, 1), lambda i: (i, 0)),
            pl.BlockSpec((_BM, 3), lambda i: (i, 0)),
            pl.BlockSpec((128, _HID), lambda i: (0, 0)),
            pl.BlockSpec((_HID, _HID), lambda i: (0, 0)),
            pl.BlockSpec((3, _HID), lambda i: (0, 0)),
        ],
        out_specs=pl.BlockSpec((_BM, _HID), lambda i: (i, 0)),
        out_shape=jax.ShapeDtypeStruct((_N, _HID), jnp.float32),
    )(x, pos, emb, w1a, w1b)


def _tc_scale(t, degp):
    def body(t_ref, degp_ref, u_ref, dinv_ref):
        deg = 1.0 + degp_ref[0, :, 0:1] + degp_ref[1, :, 0:1]
        dinv = lax.rsqrt(deg)
        u_ref[...] = dinv * t_ref[...]
        dinv_ref[...] = jnp.broadcast_to(dinv, (_BM, 16))

    return pl.pallas_call(
        body,
        grid=(_N // _BM,),
        in_specs=[
            pl.BlockSpec((_BM, _HID), lambda i: (i, 0)),
            pl.BlockSpec((_NC, _BM, _HID), lambda i: (0, i, 0)),
        ],
        out_specs=[
            pl.BlockSpec((_BM, _HID), lambda i: (i, 0)),
            pl.BlockSpec((_BM, 16), lambda i: (i, 0)),
        ],
        out_shape=[
            jax.ShapeDtypeStruct((_N, _HID), jnp.float32),
            jax.ShapeDtypeStruct((_N, 16), jnp.float32),
        ],
    )(t, degp)


def _tc_combine(a, u, dinv16, w, b):
    def body(a_ref, u_ref, dinv_ref, w_ref, b_ref, out_ref):
        dinv = dinv_ref[:, 0:1]
        h = jnp.maximum(dinv * (a_ref[0] + a_ref[1] + u_ref[...]) + b_ref[...], 0.0)
        t = jnp.dot(h, w_ref[...], preferred_element_type=jnp.float32)
        out_ref[...] = dinv * t

    return pl.pallas_call(
        body,
        grid=(_N // _BM,),
        in_specs=[
            pl.BlockSpec((_NC, _BM, _HID), lambda i: (0, i, 0)),
            pl.BlockSpec((_BM, _HID), lambda i: (i, 0)),
            pl.BlockSpec((_BM, 16), lambda i: (i, 0)),
            pl.BlockSpec((_HID, _HID), lambda i: (0, 0)),
            pl.BlockSpec((1, _HID), lambda i: (0, 0)),
        ],
        out_specs=pl.BlockSpec((_BM, _HID), lambda i: (i, 0)),
        out_shape=jax.ShapeDtypeStruct((_N, _HID), jnp.float32),
    )(a, u, dinv16, w, b)


def _tc_combine_nw(a, u, dinv16, b):
    def body(a_ref, u_ref, dinv_ref, b_ref, out_ref):
        dinv = dinv_ref[:, 0:1]
        h = jnp.maximum(dinv * (a_ref[0] + a_ref[1] + u_ref[...]) + b_ref[...], 0.0)
        out_ref[...] = dinv * h

    return pl.pallas_call(
        body,
        grid=(_N // _BM,),
        in_specs=[
            pl.BlockSpec((_NC, _BM, _HID), lambda i: (0, i, 0)),
            pl.BlockSpec((_BM, _HID), lambda i: (i, 0)),
            pl.BlockSpec((_BM, 16), lambda i: (i, 0)),
            pl.BlockSpec((1, _HID), lambda i: (0, 0)),
        ],
        out_specs=pl.BlockSpec((_BM, _HID), lambda i: (i, 0)),
        out_shape=jax.ShapeDtypeStruct((_N, _HID), jnp.float32),
    )(a, u, dinv16, b)


def _tc_heads(a, u, dinv16, wmu, bmu, wls, bls, eps):
    def body(a_ref, u_ref, dinv_ref, wmu_ref, bmu_ref, wls_ref, bls_ref, eps_ref,
             mu_ref, ls_ref, z_ref):
        dinv = dinv_ref[:, 0:1]
        sfeat = dinv * (a_ref[0] + a_ref[1] + u_ref[...])
        mu = jnp.dot(sfeat, wmu_ref[...], preferred_element_type=jnp.float32) + bmu_ref[...]
        ls = jnp.dot(sfeat, wls_ref[...], preferred_element_type=jnp.float32) + bls_ref[...]
        mu_ref[...] = mu
        ls_ref[...] = ls
        z_ref[...] = (mu + eps_ref[...] * jnp.exp(0.5 * ls)).astype(jnp.bfloat16)

    return pl.pallas_call(
        body,
        grid=(_N // _BM,),
        in_specs=[
            pl.BlockSpec((_NC, _BM, _HID), lambda i: (0, i, 0)),
            pl.BlockSpec((_BM, _HID), lambda i: (i, 0)),
            pl.BlockSpec((_BM, 16), lambda i: (i, 0)),
            pl.BlockSpec((_HID, _LAT), lambda i: (0, 0)),
            pl.BlockSpec((1, _LAT), lambda i: (0, 0)),
            pl.BlockSpec((_HID, _LAT), lambda i: (0, 0)),
            pl.BlockSpec((1, _LAT), lambda i: (0, 0)),
            pl.BlockSpec((_BM, _LAT), lambda i: (i, 0)),
        ],
        out_specs=[
            pl.BlockSpec((_BM, _LAT), lambda i: (i, 0)),
            pl.BlockSpec((_BM, _LAT), lambda i: (i, 0)),
            pl.BlockSpec((_BM, _LAT), lambda i: (i, 0)),
        ],
        out_shape=[
            jax.ShapeDtypeStruct((_N, _LAT), jnp.float32),
            jax.ShapeDtypeStruct((_N, _LAT), jnp.float32),
            jax.ShapeDtypeStruct((_N, _LAT), jnp.bfloat16),
        ],
    )(a, u, dinv16, wmu, bmu, wls, bls, eps)


def _tc_zz(z):
    def body(zi_ref, zj_ref, out_ref):
        out_ref[...] = lax.dot_general(
            zi_ref[...], zj_ref[...], (((1,), (1,)), ((), ())),
            preferred_element_type=jnp.float32)

    bz = 200
    return pl.pallas_call(
        body,
        grid=(_N // bz,),
        in_specs=[
            pl.BlockSpec((bz, _LAT), lambda i: (i, 0)),
            pl.BlockSpec((_N, _LAT), lambda i: (0, 0)),
        ],
        out_specs=pl.BlockSpec((bz, _N), lambda i: (i, 0)),
        out_shape=jax.ShapeDtypeStruct((_N, _N), jnp.float32),
    )(z, z)


def kernel(x, pos, edge_index, emb, W1, b1, W2, b2, Wmu, bmu, Wls, bls):
    src_r = edge_index[0].reshape(_NC, _NS, _NCH, _K)
    dst_r = edge_index[1].reshape(_NC, _NS, _NCH, _K)
    ones128 = jnp.ones((_K, _HID), jnp.float32)
    zeros128 = jnp.zeros((_RPT, _HID), jnp.float32)
    w1a = W1[:_HID]
    w1b = W1[_HID:]
    emb_p = jnp.pad(emb, ((0, 128 - emb.shape[0]), (0, 0)))
    eps = jax.random.normal(jax.random.key(42), (_N, _LAT), jnp.float32)

    degp = _sc_deg(dst_r, ones128, zeros128)
    t1 = _tc_embed(x, pos, emb_p, w1a, w1b)
    u1, dinv16 = _tc_scale(t1, degp)
    a1 = _sc_prop(u1, src_r, dst_r, zeros128)
    u2 = _tc_combine(a1, u1, dinv16, W2, b1.reshape(1, _HID))
    a2 = _sc_prop(u2, src_r, dst_r, zeros128)
    u3 = _tc_combine_nw(a2, u2, dinv16, b2.reshape(1, _HID))
    a3 = _sc_prop(u3, src_r, dst_r, zeros128)
    mu, ls, z = _tc_heads(a3, u3, dinv16, Wmu, bmu.reshape(1, _LAT),
                          Wls, bls.reshape(1, _LAT), eps)
    adj = _tc_zz(z)
    return adj, mu, ls
